# Initial kernel scaffold; baseline (speedup 1.0000x reference)
import jax
import jax.numpy as jnp
from jax.experimental import pallas as pl


def kernel(x, W_cat, b_cat, W_mu, b_mu, log_std, deterministic):
    raise NotImplementedError("write your pallas kernel here")


# speedup vs baseline: 3.5354x; 3.5354x over previous
import functools
import math

import jax
import jax.numpy as jnp
from jax.experimental import pallas as pl
from jax.experimental.pallas import tpu as pltpu

_NUM_HEADS = 20
_N_SC = 64
_SC_CAP = 4.0
_NOISE_EPS = 0.1
_NOISE_SCALE = 0.1
_NEG_INF = -1e10
_TILE_B = 512


def _body(x_ref, wc_ref, bc_ref, wmu_ref, bmu_ref, lstd_ref,
          rmask_ref, rand_ref, ncont_ref, out_ref, logp_ref):
    x = x_ref[...]
    logits_all = jnp.dot(x, wc_ref[...], preferred_element_type=jnp.float32)
    logits_all = logits_all + bc_ref[...]
    mean = jnp.dot(x, wmu_ref[...], preferred_element_type=jnp.float32)
    mean = mean + bmu_ref[...]

    tb = x.shape[0]
    iota64 = jax.lax.broadcasted_iota(jnp.int32, (tb, _N_SC), 1)
    iota20 = jax.lax.broadcasted_iota(jnp.int32, (tb, _NUM_HEADS), 1)

    sc_stat = jnp.zeros((tb, _N_SC), dtype=jnp.float32)
    disc_lp = jnp.zeros((tb, 1), dtype=jnp.float32)
    acts = jnp.zeros((tb, _NUM_HEADS), dtype=jnp.float32)
    rmask = rmask_ref[...]
    rand = rand_ref[...]
    for i in range(_NUM_HEADS):
        l = logits_all[:, i * _N_SC:(i + 1) * _N_SC]
        lm = jnp.where(sc_stat < _SC_CAP, l, _NEG_INF)
        mx = jnp.max(lm, axis=-1, keepdims=True)
        amax = jnp.min(jnp.where(lm == mx, iota64, _N_SC), axis=-1,
                       keepdims=True)
        logp = -jnp.log(jnp.sum(jnp.exp(lm - mx), axis=-1, keepdims=True))
        rm = rmask[:, i:i + 1]
        act_f = rm * rand[:, i:i + 1] + (1.0 - rm) * amax.astype(jnp.float32)
        chosen = act_f.astype(jnp.int32)
        sc_stat = sc_stat + (iota64 == chosen).astype(jnp.float32)
        disc_lp = disc_lp + logp
        acts = jnp.where(iota20 == i, act_f, acts)

    cont = mean + ncont_ref[...] * _NOISE_SCALE
    dlt = cont - mean
    lstd = lstd_ref[...]
    std = jnp.exp(lstd)
    cont_lp = jnp.sum(
        -(dlt * dlt) / (2.0 * std * std) - lstd - 0.5 * math.log(2.0 * math.pi),
        axis=-1, keepdims=True)
    out_ref[...] = jnp.concatenate([acts, cont], axis=1)
    logp_ref[...] = disc_lp + cont_lp


def _noise(batch):
    key = jax.random.key(42)
    rmask_cols, rand_cols = [], []
    for i in range(_NUM_HEADS):
        k1 = jax.random.fold_in(key, 2 * i)
        k2 = jax.random.fold_in(key, 2 * i + 1)
        rmask_cols.append(
            (jax.random.uniform(k1, (batch, 1)) < _NOISE_EPS).astype(jnp.float32))
        rand_cols.append(
            jax.random.randint(k2, (batch, 1), 0, _N_SC).astype(jnp.float32))
    kc = jax.random.fold_in(key, 999)
    ncont = jax.random.normal(kc, (batch, _NUM_HEADS), dtype=jnp.float32)
    return (jnp.concatenate(rmask_cols, axis=1),
            jnp.concatenate(rand_cols, axis=1), ncont)


def kernel(x, W_cat, b_cat, W_mu, b_mu, log_std, deterministic):
    del deterministic
    batch, d = x.shape
    wc = jnp.transpose(W_cat, (1, 0, 2)).reshape(d, _NUM_HEADS * _N_SC)
    bc = b_cat.reshape(1, _NUM_HEADS * _N_SC)
    rmask, rand, ncont = _noise(batch)

    tb = _TILE_B
    grid = (batch // tb,)
    out, logp = pl.pallas_call(
        _body,
        grid=grid,
        in_specs=[
            pl.BlockSpec((tb, d), lambda i: (i, 0)),
            pl.BlockSpec((d, _NUM_HEADS * _N_SC), lambda i: (0, 0)),
            pl.BlockSpec((1, _NUM_HEADS * _N_SC), lambda i: (0, 0)),
            pl.BlockSpec((d, _NUM_HEADS), lambda i: (0, 0)),
            pl.BlockSpec((1, _NUM_HEADS), lambda i: (0, 0)),
            pl.BlockSpec((1, _NUM_HEADS), lambda i: (0, 0)),
            pl.BlockSpec((tb, _NUM_HEADS), lambda i: (i, 0)),
            pl.BlockSpec((tb, _NUM_HEADS), lambda i: (i, 0)),
            pl.BlockSpec((tb, _NUM_HEADS), lambda i: (i, 0)),
        ],
        out_specs=[
            pl.BlockSpec((tb, 2 * _NUM_HEADS), lambda i: (i, 0)),
            pl.BlockSpec((tb, 1), lambda i: (i, 0)),
        ],
        out_shape=[
            jax.ShapeDtypeStruct((batch, 2 * _NUM_HEADS), jnp.float32),
            jax.ShapeDtypeStruct((batch, 1), jnp.float32),
        ],
        compiler_params=pltpu.CompilerParams(
            dimension_semantics=("parallel",)),
    )(x, wc, bc, W_mu, b_mu.reshape(1, _NUM_HEADS),
      log_std.reshape(1, _NUM_HEADS), rmask, rand, ncont)
    return out, logp


# speedup vs baseline: 8.8469x; 2.5024x over previous
import functools
import math

import jax
import jax.numpy as jnp
from jax.experimental import pallas as pl
from jax.experimental.pallas import tpu as pltpu

_NUM_HEADS = 20
_N_SC = 64
_SC_CAP = 4.0
_NOISE_EPS = 0.1
_NOISE_SCALE = 0.1
_NEG_INF = -1e10
_TILE_B = 512


def _body(x_ref, wc_ref, bc_ref, wmu_ref, bmu_ref, lstd_ref,
          rmask_ref, rand_ref, ncont_ref, out_ref, logp_ref):
    x = x_ref[...]
    logits_all = jnp.dot(x, wc_ref[...], preferred_element_type=jnp.float32)
    logits_all = logits_all + bc_ref[...]
    mean = jnp.dot(x, wmu_ref[...], preferred_element_type=jnp.float32)
    mean = mean + bmu_ref[...]

    tb = x.shape[0]
    iota64 = jax.lax.broadcasted_iota(jnp.int32, (tb, _N_SC), 1)
    iota20 = jax.lax.broadcasted_iota(jnp.int32, (tb, _NUM_HEADS), 1)

    sc_stat = jnp.zeros((tb, _N_SC), dtype=jnp.float32)
    disc_lp = jnp.zeros((tb, 1), dtype=jnp.float32)
    acts = jnp.zeros((tb, _NUM_HEADS), dtype=jnp.float32)
    rmask = rmask_ref[...]
    rand = rand_ref[...]
    for i in range(_NUM_HEADS):
        l = logits_all[:, i * _N_SC:(i + 1) * _N_SC]
        lm = jnp.where(sc_stat < _SC_CAP, l, _NEG_INF)
        mx = jnp.max(lm, axis=-1, keepdims=True)
        amax = jnp.min(jnp.where(lm == mx, iota64, _N_SC), axis=-1,
                       keepdims=True)
        logp = -jnp.log(jnp.sum(jnp.exp(lm - mx), axis=-1, keepdims=True))
        rm = rmask[:, i:i + 1]
        act_f = rm * rand[:, i:i + 1] + (1.0 - rm) * amax.astype(jnp.float32)
        chosen = act_f.astype(jnp.int32)
        sc_stat = sc_stat + (iota64 == chosen).astype(jnp.float32)
        disc_lp = disc_lp + logp
        acts = jnp.where(iota20 == i, act_f, acts)

    cont = mean + ncont_ref[...] * _NOISE_SCALE
    dlt = cont - mean
    lstd = lstd_ref[...]
    std = jnp.exp(lstd)
    cont_lp = jnp.sum(
        -(dlt * dlt) / (2.0 * std * std) - lstd - 0.5 * math.log(2.0 * math.pi),
        axis=-1, keepdims=True)
    out_ref[...] = jnp.concatenate([acts, cont], axis=1)
    logp_ref[...] = disc_lp + cont_lp


def _noise(batch):
    key = jax.random.key(42)
    steps = jnp.arange(_NUM_HEADS)
    ks0 = jax.vmap(lambda i: jax.random.fold_in(key, i))(2 * steps)
    ks1 = jax.vmap(lambda i: jax.random.fold_in(key, i))(2 * steps + 1)
    rmask = jax.vmap(lambda k: jax.random.uniform(k, (batch,)))(ks0)
    rmask = (rmask < _NOISE_EPS).astype(jnp.float32).T
    rand = jax.vmap(lambda k: jax.random.randint(k, (batch,), 0, _N_SC))(ks1)
    rand = rand.astype(jnp.float32).T
    kc = jax.random.fold_in(key, 999)
    ncont = jax.random.normal(kc, (batch, _NUM_HEADS), dtype=jnp.float32)
    return rmask, rand, ncont


def kernel(x, W_cat, b_cat, W_mu, b_mu, log_std, deterministic):
    del deterministic
    batch, d = x.shape
    wc = jnp.transpose(W_cat, (1, 0, 2)).reshape(d, _NUM_HEADS * _N_SC)
    bc = b_cat.reshape(1, _NUM_HEADS * _N_SC)
    rmask, rand, ncont = _noise(batch)

    tb = _TILE_B
    grid = (batch // tb,)
    out, logp = pl.pallas_call(
        _body,
        grid=grid,
        in_specs=[
            pl.BlockSpec((tb, d), lambda i: (i, 0)),
            pl.BlockSpec((d, _NUM_HEADS * _N_SC), lambda i: (0, 0)),
            pl.BlockSpec((1, _NUM_HEADS * _N_SC), lambda i: (0, 0)),
            pl.BlockSpec((d, _NUM_HEADS), lambda i: (0, 0)),
            pl.BlockSpec((1, _NUM_HEADS), lambda i: (0, 0)),
            pl.BlockSpec((1, _NUM_HEADS), lambda i: (0, 0)),
            pl.BlockSpec((tb, _NUM_HEADS), lambda i: (i, 0)),
            pl.BlockSpec((tb, _NUM_HEADS), lambda i: (i, 0)),
            pl.BlockSpec((tb, _NUM_HEADS), lambda i: (i, 0)),
        ],
        out_specs=[
            pl.BlockSpec((tb, 2 * _NUM_HEADS), lambda i: (i, 0)),
            pl.BlockSpec((tb, 1), lambda i: (i, 0)),
        ],
        out_shape=[
            jax.ShapeDtypeStruct((batch, 2 * _NUM_HEADS), jnp.float32),
            jax.ShapeDtypeStruct((batch, 1), jnp.float32),
        ],
        compiler_params=pltpu.CompilerParams(
            dimension_semantics=("parallel",)),
    )(x, wc, bc, W_mu, b_mu.reshape(1, _NUM_HEADS),
      log_std.reshape(1, _NUM_HEADS), rmask, rand, ncont)
    return out, logp
